# Initial kernel scaffold; baseline (speedup 1.0000x reference)
import jax
import jax.numpy as jnp
from jax.experimental import pallas as pl


def kernel(x, edge_index, W1, att_src1, att_dst1, b1, W2, att_src2, att_dst2, b2):
    raise NotImplementedError("write your pallas kernel here")


# speedup vs baseline: 16.5867x; 16.5867x over previous
import functools

import jax
import jax.numpy as jnp
from jax import lax
from jax.experimental import pallas as pl
from jax.experimental.pallas import tpu as pltpu
from jax.experimental.pallas import tpu_sc as plsc

NC = 2
NS = 16
NW = NC * NS
LANES = 16
B_EDGE = 16
ROW_BLK = 512


def _tc_embed(x_pad, W, att_src, att_dst, heads, C):
    NP = x_pad.shape[0]
    D = x_pad.shape[1]
    HC = W.shape[1]

    def body(x_ref, w_ref, as_ref, ad_ref, h_ref, ats_ref, atd_ref):
        h = jnp.dot(x_ref[...], w_ref[...], preferred_element_type=jnp.float32)
        h_ref[...] = h
        ats_ref[...] = jnp.stack(
            [jnp.sum(h[:, i * C:(i + 1) * C] * as_ref[i, :][None, :], axis=1)
             for i in range(heads)])
        atd_ref[...] = jnp.stack(
            [jnp.sum(h[:, i * C:(i + 1) * C] * ad_ref[i, :][None, :], axis=1)
             for i in range(heads)])

    return pl.pallas_call(
        body,
        grid=(NP // ROW_BLK,),
        in_specs=[
            pl.BlockSpec((ROW_BLK, D), lambda i: (i, 0)),
            pl.BlockSpec((D, HC), lambda i: (0, 0)),
            pl.BlockSpec((heads, C), lambda i: (0, 0)),
            pl.BlockSpec((heads, C), lambda i: (0, 0)),
        ],
        out_specs=[
            pl.BlockSpec((ROW_BLK, HC), lambda i: (i, 0)),
            pl.BlockSpec((heads, ROW_BLK), lambda i: (0, i)),
            pl.BlockSpec((heads, ROW_BLK), lambda i: (0, i)),
        ],
        out_shape=[
            jax.ShapeDtypeStruct((NP, HC), jnp.float32),
            jax.ShapeDtypeStruct((heads, NP), jnp.float32),
            jax.ShapeDtypeStruct((heads, NP), jnp.float32),
        ],
    )(x_pad, W, att_src, att_dst)


def _tc_reduce_denom(dpart2d):
    G, M = dpart2d.shape

    def body(dp_ref, d_ref):
        d_ref[...] = jnp.sum(dp_ref[...], axis=0, keepdims=True)

    return pl.pallas_call(
        body,
        grid=(1,),
        in_specs=[pl.BlockSpec((G, M), lambda i: (0, 0))],
        out_specs=pl.BlockSpec((1, M), lambda i: (0, 0)),
        out_shape=jax.ShapeDtypeStruct((1, M), jnp.float32),
    )(dpart2d)


def _tc_combine(pa, pb, bias2d, apply_elu):
    NP, C = pa.shape

    def body(pa_ref, pb_ref, b_ref, o_ref):
        v = pa_ref[...] + pb_ref[...] + b_ref[...]
        if apply_elu:
            v = jnp.where(v > 0, v, jnp.exp(v) - 1.0)
        o_ref[...] = v

    return pl.pallas_call(
        body,
        grid=(NP // ROW_BLK,),
        in_specs=[
            pl.BlockSpec((ROW_BLK, C), lambda i: (i, 0)),
            pl.BlockSpec((ROW_BLK, C), lambda i: (i, 0)),
            pl.BlockSpec((1, C), lambda i: (0, 0)),
        ],
        out_specs=pl.BlockSpec((ROW_BLK, C), lambda i: (i, 0)),
        out_shape=jax.ShapeDtypeStruct((NP, C), jnp.float32),
    )(pa, pb, bias2d)


def _sc_pass1(asrc_t, adst_t, src_f, dst_f, heads, NP, EP):
    EPT = EP // (NW // heads)
    NGRP = NW // heads
    mesh = plsc.VectorSubcoreMesh(
        core_axis_name="c", subcore_axis_name="s",
        num_cores=NC, num_subcores=NS)

    @functools.partial(
        pl.kernel, mesh=mesh,
        compiler_params=pltpu.CompilerParams(needs_layout_passes=False),
        out_type=(
            jax.ShapeDtypeStruct((heads * EP,), jnp.float32),
            jax.ShapeDtypeStruct((NGRP * heads * NP,), jnp.float32),
        ),
        scratch_types=[
            pltpu.VMEM((NP,), jnp.float32),
            pltpu.VMEM((NP,), jnp.float32),
            pltpu.VMEM((NP,), jnp.float32),
            pltpu.VMEM((EPT,), jnp.int32),
            pltpu.VMEM((EPT,), jnp.int32),
            pltpu.VMEM((EPT,), jnp.float32),
        ],
    )
    def k(asrc_hbm, adst_hbm, src_hbm, dst_hbm, p_hbm, dpart_hbm,
          asrc_v, adst_v, den_v, src_v, dst_v, p_v):
        wid = lax.axis_index("s") * NC + lax.axis_index("c")
        g = wid // heads
        hd = wid % heads
        pltpu.sync_copy(asrc_hbm.at[pl.ds(hd * NP, NP)], asrc_v)
        pltpu.sync_copy(adst_hbm.at[pl.ds(hd * NP, NP)], adst_v)
        e0 = g * EPT
        pltpu.sync_copy(src_hbm.at[pl.ds(e0, EPT)], src_v)
        pltpu.sync_copy(dst_hbm.at[pl.ds(e0, EPT)], dst_v)

        def zbody(i, carry):
            den_v[pl.ds(i * LANES, LANES)] = jnp.zeros((LANES,), jnp.float32)
            return carry
        lax.fori_loop(0, NP // LANES, zbody, 0)

        def ebody(i, carry):
            s16 = src_v[pl.ds(i * LANES, LANES)]
            d16 = dst_v[pl.ds(i * LANES, LANES)]
            a_s = plsc.load_gather(asrc_v, [s16])
            a_d = plsc.load_gather(adst_v, [d16])
            t = a_s + a_d
            t = jnp.where(t >= 0.0, t, 0.2 * t)
            p = jnp.exp(t)
            p_v[pl.ds(i * LANES, LANES)] = p
            plsc.addupdate_scatter(den_v, [d16], p)
            return carry
        lax.fori_loop(0, EPT // LANES, ebody, 0)

        pltpu.sync_copy(p_v, p_hbm.at[pl.ds(hd * EP + e0, EPT)])
        pltpu.sync_copy(den_v, dpart_hbm.at[pl.ds((g * heads + hd) * NP, NP)])

    return k(asrc_t, adst_t, src_f, dst_f)


def _sc_alpha(p, den_f, dst_f, heads, NP, EP):
    EPT = EP // (NW // heads)
    mesh = plsc.VectorSubcoreMesh(
        core_axis_name="c", subcore_axis_name="s",
        num_cores=NC, num_subcores=NS)

    @functools.partial(
        pl.kernel, mesh=mesh,
        compiler_params=pltpu.CompilerParams(needs_layout_passes=False),
        out_type=jax.ShapeDtypeStruct((heads * EP,), jnp.float32),
        scratch_types=[
            pltpu.VMEM((NP,), jnp.float32),
            pltpu.VMEM((EPT,), jnp.int32),
            pltpu.VMEM((EPT,), jnp.float32),
        ],
    )
    def k(p_hbm, den_hbm, dst_hbm, alpha_hbm, den_v, dst_v, pa_v):
        wid = lax.axis_index("s") * NC + lax.axis_index("c")
        g = wid // heads
        hd = wid % heads
        e0 = g * EPT
        pltpu.sync_copy(den_hbm.at[pl.ds(hd * NP, NP)], den_v)
        pltpu.sync_copy(dst_hbm.at[pl.ds(e0, EPT)], dst_v)
        pltpu.sync_copy(p_hbm.at[pl.ds(hd * EP + e0, EPT)], pa_v)

        def ebody(i, carry):
            d16 = dst_v[pl.ds(i * LANES, LANES)]
            dn = plsc.load_gather(den_v, [d16])
            pv = pa_v[pl.ds(i * LANES, LANES)]
            pa_v[pl.ds(i * LANES, LANES)] = (
                (1.0 / heads) * pv / (dn + 1e-16))
            return carry
        lax.fori_loop(0, EPT // LANES, ebody, 0)
        pltpu.sync_copy(pa_v, alpha_hbm.at[pl.ds(hd * EP + e0, EPT)])

    return k(p, den_f, dst_f)


def _sc_pass2(h, src_f, dst_f, alpha, zeros_np, heads, C, NP, EP):
    EPT = EP // NW
    SB = 128
    GB = 32
    NSC = EPT // SB
    RT = NP // NS
    mesh = plsc.VectorSubcoreMesh(
        core_axis_name="c", subcore_axis_name="s",
        num_cores=NC, num_subcores=NS)

    @functools.partial(
        pl.kernel, mesh=mesh,
        compiler_params=pltpu.CompilerParams(needs_layout_passes=False),
        out_type=jax.ShapeDtypeStruct((NC, NP, C), jnp.float32),
        scratch_types=[
            pltpu.VMEM((SB,), jnp.int32),
            pltpu.VMEM((SB,), jnp.int32),
            pltpu.VMEM((heads * SB,), jnp.float32),
            pltpu.VMEM((GB, heads * C), jnp.float32),
            pltpu.VMEM((SB, C), jnp.float32),
            pltpu.SemaphoreType.DMA,
            pltpu.VMEM_SHARED((NP, C), jnp.float32),
        ],
    )
    def k(h_hbm, src_hbm, dst_hbm, alpha_hbm, z_hbm, outp_hbm,
          src_v, dst_v, alpha_v, rows_v, msg_v, sem, acc):
        cid = lax.axis_index("c")
        sid = lax.axis_index("s")
        wid = sid * NC + cid
        e0 = wid * EPT
        pltpu.sync_copy(z_hbm.at[pl.ds(sid * RT, RT)],
                        acc.at[pl.ds(sid * RT, RT)])
        plsc.subcore_barrier()

        def schunk(s, carry):
            base = e0 + s * SB
            pltpu.sync_copy(src_hbm.at[pl.ds(base, SB)], src_v)
            pltpu.sync_copy(dst_hbm.at[pl.ds(base, SB)], dst_v)
            for hd in range(heads):
                pltpu.sync_copy(alpha_hbm.at[pl.ds(hd * EP + base, SB)],
                                alpha_v.at[pl.ds(hd * SB, SB)])

            def gchunk(g, carry2):
                pltpu.async_copy(
                    h_hbm.at[src_v.at[pl.ds(g * GB, GB)]], rows_v, sem).wait()

                def ebody(e, carry3):
                    e_splat = jnp.full((LANES,), 0, jnp.int32) + e
                    avec = [
                        plsc.load_gather(
                            alpha_v, [e_splat + g * GB + hd * SB])
                        for hd in range(heads)]
                    m = g * GB + e
                    for jj in range(C // LANES):
                        accv = avec[0] * rows_v[e, pl.ds(jj * LANES, LANES)]
                        for hd in range(1, heads):
                            accv = accv + avec[hd] * rows_v[
                                e, pl.ds(hd * C + jj * LANES, LANES)]
                        msg_v[m, pl.ds(jj * LANES, LANES)] = accv
                    return carry3
                lax.fori_loop(0, GB, ebody, 0)
                return carry2
            lax.fori_loop(0, SB // GB, gchunk, 0)
            pltpu.sync_copy(msg_v, acc.at[dst_v], add=True)
            return carry
        lax.fori_loop(0, NSC, schunk, 0)

        plsc.subcore_barrier()
        pltpu.sync_copy(acc.at[pl.ds(sid * RT, RT)],
                        outp_hbm.at[cid, pl.ds(sid * RT, RT)])

    return k(h, src_f, dst_f, alpha, zeros_np)


def kernel(x, edge_index, W1, att_src1, att_dst1, b1,
           W2, att_src2, att_dst2, b2):
    N, D = x.shape
    E = edge_index.shape[1]
    heads, C = att_src1.shape

    NP = ((N + 1 + ROW_BLK - 1) // ROW_BLK) * ROW_BLK
    E2 = E + N
    EPGRAN = NW * 128
    EP = ((E2 + EPGRAN - 1) // EPGRAN) * EPGRAN

    loop = jnp.arange(N, dtype=edge_index.dtype)
    src = jnp.concatenate([edge_index[0], loop])
    dst = jnp.concatenate([edge_index[1], loop])
    src_f = jnp.pad(src, (0, EP - E2))
    dst_f = jnp.pad(dst, (0, EP - E2), constant_values=N)
    x_pad = jnp.pad(x, ((0, NP - N), (0, 0)))
    zeros_np = jnp.zeros((NP, C), dtype=jnp.float32)

    def layer(xp, W, att_s, att_d):
        h, asrc_t, adst_t = _tc_embed(xp, W, att_s, att_d, heads, C)
        p, dpart = _sc_pass1(asrc_t.reshape(-1), adst_t.reshape(-1),
                             src_f, dst_f, heads, NP, EP)
        den = _tc_reduce_denom(dpart.reshape(NW // heads, heads * NP))
        den_f = den.reshape(heads * NP)
        alpha = _sc_alpha(p, den_f, dst_f, heads, NP, EP)
        outp = _sc_pass2(h, src_f, dst_f, alpha, zeros_np, heads, C, NP, EP)
        return outp[0], outp[1]

    pa1, pb1 = layer(x_pad, W1, att_src1, att_dst1)
    x2 = _tc_combine(pa1, pb1, b1.reshape(1, C), apply_elu=True)
    pa2, pb2 = layer(x2, W2, att_src2, att_dst2)
    out_full = _tc_combine(pa2, pb2, b2.reshape(1, C), apply_elu=False)
    return out_full[:N]


# speedup vs baseline: 23.7571x; 1.4323x over previous
import functools

import jax
import jax.numpy as jnp
from jax import lax
from jax.experimental import pallas as pl
from jax.experimental.pallas import tpu as pltpu
from jax.experimental.pallas import tpu_sc as plsc

NC = 2
NS = 16
NW = NC * NS
LANES = 16
B_EDGE = 16
ROW_BLK = 512


def _tc_embed(x_pad, W, att_src, att_dst, heads, C):
    NP = x_pad.shape[0]
    D = x_pad.shape[1]
    HC = W.shape[1]

    def body(x_ref, w_ref, as_ref, ad_ref, h_ref, ats_ref, atd_ref):
        h = jnp.dot(x_ref[...], w_ref[...], preferred_element_type=jnp.float32)
        h_ref[...] = h
        ats_ref[...] = jnp.stack(
            [jnp.sum(h[:, i * C:(i + 1) * C] * as_ref[i, :][None, :], axis=1)
             for i in range(heads)])
        atd_ref[...] = jnp.stack(
            [jnp.sum(h[:, i * C:(i + 1) * C] * ad_ref[i, :][None, :], axis=1)
             for i in range(heads)])

    return pl.pallas_call(
        body,
        grid=(NP // ROW_BLK,),
        in_specs=[
            pl.BlockSpec((ROW_BLK, D), lambda i: (i, 0)),
            pl.BlockSpec((D, HC), lambda i: (0, 0)),
            pl.BlockSpec((heads, C), lambda i: (0, 0)),
            pl.BlockSpec((heads, C), lambda i: (0, 0)),
        ],
        out_specs=[
            pl.BlockSpec((ROW_BLK, HC), lambda i: (i, 0)),
            pl.BlockSpec((heads, ROW_BLK), lambda i: (0, i)),
            pl.BlockSpec((heads, ROW_BLK), lambda i: (0, i)),
        ],
        out_shape=[
            jax.ShapeDtypeStruct((NP, HC), jnp.float32),
            jax.ShapeDtypeStruct((heads, NP), jnp.float32),
            jax.ShapeDtypeStruct((heads, NP), jnp.float32),
        ],
    )(x_pad, W, att_src, att_dst)


def _tc_reduce_denom(dpart2d):
    G, M = dpart2d.shape

    def body(dp_ref, d_ref):
        d_ref[...] = jnp.sum(dp_ref[...], axis=0, keepdims=True)

    return pl.pallas_call(
        body,
        grid=(1,),
        in_specs=[pl.BlockSpec((G, M), lambda i: (0, 0))],
        out_specs=pl.BlockSpec((1, M), lambda i: (0, 0)),
        out_shape=jax.ShapeDtypeStruct((1, M), jnp.float32),
    )(dpart2d)


def _tc_combine(pa, pb, bias2d, apply_elu):
    NP, C = pa.shape

    def body(pa_ref, pb_ref, b_ref, o_ref):
        v = pa_ref[...] + pb_ref[...] + b_ref[...]
        if apply_elu:
            v = jnp.where(v > 0, v, jnp.exp(v) - 1.0)
        o_ref[...] = v

    return pl.pallas_call(
        body,
        grid=(NP // ROW_BLK,),
        in_specs=[
            pl.BlockSpec((ROW_BLK, C), lambda i: (i, 0)),
            pl.BlockSpec((ROW_BLK, C), lambda i: (i, 0)),
            pl.BlockSpec((1, C), lambda i: (0, 0)),
        ],
        out_specs=pl.BlockSpec((ROW_BLK, C), lambda i: (i, 0)),
        out_shape=jax.ShapeDtypeStruct((NP, C), jnp.float32),
    )(pa, pb, bias2d)


def _sc_pass1(asrc_t, adst_t, src_f, dst_f, heads, NP, EP):
    EPT = EP // (NW // heads)
    NGRP = NW // heads
    mesh = plsc.VectorSubcoreMesh(
        core_axis_name="c", subcore_axis_name="s",
        num_cores=NC, num_subcores=NS)

    @functools.partial(
        pl.kernel, mesh=mesh,
        compiler_params=pltpu.CompilerParams(needs_layout_passes=False),
        out_type=(
            jax.ShapeDtypeStruct((heads * EP,), jnp.float32),
            jax.ShapeDtypeStruct((NGRP * heads * NP,), jnp.float32),
        ),
        scratch_types=[
            pltpu.VMEM((NP,), jnp.float32),
            pltpu.VMEM((NP,), jnp.float32),
            pltpu.VMEM((NP,), jnp.float32),
            pltpu.VMEM((EPT,), jnp.int32),
            pltpu.VMEM((EPT,), jnp.int32),
            pltpu.VMEM((EPT,), jnp.float32),
        ],
    )
    def k(asrc_hbm, adst_hbm, src_hbm, dst_hbm, p_hbm, dpart_hbm,
          asrc_v, adst_v, den_v, src_v, dst_v, p_v):
        wid = lax.axis_index("s") * NC + lax.axis_index("c")
        g = wid // heads
        hd = wid % heads
        pltpu.sync_copy(asrc_hbm.at[pl.ds(hd * NP, NP)], asrc_v)
        pltpu.sync_copy(adst_hbm.at[pl.ds(hd * NP, NP)], adst_v)
        e0 = g * EPT
        pltpu.sync_copy(src_hbm.at[pl.ds(e0, EPT)], src_v)
        pltpu.sync_copy(dst_hbm.at[pl.ds(e0, EPT)], dst_v)

        def zbody(i, carry):
            den_v[pl.ds(i * LANES, LANES)] = jnp.zeros((LANES,), jnp.float32)
            return carry
        lax.fori_loop(0, NP // LANES, zbody, 0)

        def ebody(i, carry):
            s16 = src_v[pl.ds(i * LANES, LANES)]
            d16 = dst_v[pl.ds(i * LANES, LANES)]
            a_s = plsc.load_gather(asrc_v, [s16])
            a_d = plsc.load_gather(adst_v, [d16])
            t = a_s + a_d
            t = jnp.where(t >= 0.0, t, 0.2 * t)
            p = jnp.exp(t)
            p_v[pl.ds(i * LANES, LANES)] = p
            plsc.addupdate_scatter(den_v, [d16], p)
            return carry
        lax.fori_loop(0, EPT // LANES, ebody, 0)

        pltpu.sync_copy(p_v, p_hbm.at[pl.ds(hd * EP + e0, EPT)])
        pltpu.sync_copy(den_v, dpart_hbm.at[pl.ds((g * heads + hd) * NP, NP)])

    return k(asrc_t, adst_t, src_f, dst_f)


def _sc_alpha(p, den_f, dst_f, heads, NP, EP):
    EPT = EP // (NW // heads)
    mesh = plsc.VectorSubcoreMesh(
        core_axis_name="c", subcore_axis_name="s",
        num_cores=NC, num_subcores=NS)

    @functools.partial(
        pl.kernel, mesh=mesh,
        compiler_params=pltpu.CompilerParams(needs_layout_passes=False),
        out_type=jax.ShapeDtypeStruct((heads * EP,), jnp.float32),
        scratch_types=[
            pltpu.VMEM((NP,), jnp.float32),
            pltpu.VMEM((EPT,), jnp.int32),
            pltpu.VMEM((EPT,), jnp.float32),
        ],
    )
    def k(p_hbm, den_hbm, dst_hbm, alpha_hbm, den_v, dst_v, pa_v):
        wid = lax.axis_index("s") * NC + lax.axis_index("c")
        g = wid // heads
        hd = wid % heads
        e0 = g * EPT
        pltpu.sync_copy(den_hbm.at[pl.ds(hd * NP, NP)], den_v)
        pltpu.sync_copy(dst_hbm.at[pl.ds(e0, EPT)], dst_v)
        pltpu.sync_copy(p_hbm.at[pl.ds(hd * EP + e0, EPT)], pa_v)

        def ebody(i, carry):
            d16 = dst_v[pl.ds(i * LANES, LANES)]
            dn = plsc.load_gather(den_v, [d16])
            pv = pa_v[pl.ds(i * LANES, LANES)]
            pa_v[pl.ds(i * LANES, LANES)] = (
                (1.0 / heads) * pv / (dn + 1e-16))
            return carry
        lax.fori_loop(0, EPT // LANES, ebody, 0)
        pltpu.sync_copy(pa_v, alpha_hbm.at[pl.ds(hd * EP + e0, EPT)])

    return k(p, den_f, dst_f)


def _sc_pass2(h, src_f, dst_f, alpha, zeros_np, heads, C, NP, EP):
    EPT = EP // NW
    SB = 128
    GB = 16
    NSC = EPT // SB
    RT = NP // NS
    mesh = plsc.VectorSubcoreMesh(
        core_axis_name="c", subcore_axis_name="s",
        num_cores=NC, num_subcores=NS)

    @functools.partial(
        pl.kernel, mesh=mesh,
        compiler_params=pltpu.CompilerParams(needs_layout_passes=False),
        out_type=jax.ShapeDtypeStruct((NC, NP, C), jnp.float32),
        scratch_types=[
            pltpu.VMEM((SB,), jnp.int32),
            pltpu.VMEM((SB,), jnp.int32),
            pltpu.VMEM((heads * SB,), jnp.float32),
            pltpu.VMEM((GB, heads * C), jnp.float32),
            pltpu.VMEM((GB, heads * C), jnp.float32),
            pltpu.VMEM((SB, C), jnp.float32),
            pltpu.SemaphoreType.DMA,
            pltpu.SemaphoreType.DMA,
            pltpu.SemaphoreType.DMA,
            pltpu.VMEM_SHARED((NP, C), jnp.float32),
        ],
    )
    def k(h_hbm, src_hbm, dst_hbm, alpha_hbm, z_hbm, outp_hbm,
          src_v, dst_v, alpha_v, rows_a, rows_b, msg_v,
          sem_a, sem_b, sem_i, acc):
        cid = lax.axis_index("c")
        sid = lax.axis_index("s")
        wid = sid * NC + cid
        e0 = wid * EPT
        pltpu.sync_copy(z_hbm.at[pl.ds(sid * RT, RT)],
                        acc.at[pl.ds(sid * RT, RT)])
        plsc.subcore_barrier()

        NG = SB // GB

        def schunk(s, carry):
            base = e0 + s * SB
            cp_src = pltpu.async_copy(
                src_hbm.at[pl.ds(base, SB)], src_v, sem_i)
            cp_rest = [pltpu.async_copy(
                dst_hbm.at[pl.ds(base, SB)], dst_v, sem_i)]
            for hd in range(heads):
                cp_rest.append(pltpu.async_copy(
                    alpha_hbm.at[pl.ds(hd * EP + base, SB)],
                    alpha_v.at[pl.ds(hd * SB, SB)], sem_i))
            cp_src.wait()

            rows = [rows_a, rows_b]
            sems = [sem_a, sem_b]
            cps = [None, None]
            cps[0] = pltpu.async_copy(
                h_hbm.at[src_v.at[pl.ds(0, GB)]], rows_a, sem_a)
            for cp in cp_rest:
                cp.wait()
            for g in range(NG):
                if g + 1 < NG:
                    cps[(g + 1) % 2] = pltpu.async_copy(
                        h_hbm.at[src_v.at[pl.ds((g + 1) * GB, GB)]],
                        rows[(g + 1) % 2], sems[(g + 1) % 2])
                cps[g % 2].wait()
                buf = rows[g % 2]

                def ebody(e, carry3, g=g, buf=buf):
                    e_splat = jnp.full((LANES,), 0, jnp.int32) + e
                    avec = [
                        plsc.load_gather(
                            alpha_v, [e_splat + g * GB + hd * SB])
                        for hd in range(heads)]
                    m = g * GB + e
                    for jj in range(C // LANES):
                        accv = avec[0] * buf[e, pl.ds(jj * LANES, LANES)]
                        for hd in range(1, heads):
                            accv = accv + avec[hd] * buf[
                                e, pl.ds(hd * C + jj * LANES, LANES)]
                        msg_v[m, pl.ds(jj * LANES, LANES)] = accv
                    return carry3
                lax.fori_loop(0, GB, ebody, 0)
            pltpu.sync_copy(msg_v, acc.at[dst_v], add=True)
            return carry
        lax.fori_loop(0, NSC, schunk, 0)

        plsc.subcore_barrier()
        pltpu.sync_copy(acc.at[pl.ds(sid * RT, RT)],
                        outp_hbm.at[cid, pl.ds(sid * RT, RT)])

    return k(h, src_f, dst_f, alpha, zeros_np)


def kernel(x, edge_index, W1, att_src1, att_dst1, b1,
           W2, att_src2, att_dst2, b2):
    N, D = x.shape
    E = edge_index.shape[1]
    heads, C = att_src1.shape

    NP = ((N + 1 + ROW_BLK - 1) // ROW_BLK) * ROW_BLK
    E2 = E + N
    EPGRAN = NW * 128
    EP = ((E2 + EPGRAN - 1) // EPGRAN) * EPGRAN

    loop = jnp.arange(N, dtype=edge_index.dtype)
    src = jnp.concatenate([edge_index[0], loop])
    dst = jnp.concatenate([edge_index[1], loop])
    src_f = jnp.pad(src, (0, EP - E2))
    dst_f = jnp.pad(dst, (0, EP - E2), constant_values=N)
    x_pad = jnp.pad(x, ((0, NP - N), (0, 0)))
    zeros_np = jnp.zeros((NP, C), dtype=jnp.float32)

    def layer(xp, W, att_s, att_d):
        h, asrc_t, adst_t = _tc_embed(xp, W, att_s, att_d, heads, C)
        p, dpart = _sc_pass1(asrc_t.reshape(-1), adst_t.reshape(-1),
                             src_f, dst_f, heads, NP, EP)
        den = _tc_reduce_denom(dpart.reshape(NW // heads, heads * NP))
        den_f = den.reshape(heads * NP)
        alpha = _sc_alpha(p, den_f, dst_f, heads, NP, EP)
        outp = _sc_pass2(h, src_f, dst_f, alpha, zeros_np, heads, C, NP, EP)
        return outp[0], outp[1]

    pa1, pb1 = layer(x_pad, W1, att_src1, att_dst1)
    x2 = _tc_combine(pa1, pb1, b1.reshape(1, C), apply_elu=True)
    pa2, pb2 = layer(x2, W2, att_src2, att_dst2)
    out_full = _tc_combine(pa2, pb2, b2.reshape(1, C), apply_elu=False)
    return out_full[:N]
